# baseline (device time: 33743 ns/iter reference)
import jax
import jax.numpy as jnp
from jax import lax
from jax.experimental import pallas as pl
from jax.experimental.pallas import tpu as pltpu

N_DEV = 16
N_EXPERTS = 32
CAPACITY = 102.0
GROUP = 4
STRIDE = N_DEV // GROUP
PACK_ROWS = 272


def kernel(x, router_W, route_idx, expert_W):
    m, d = x.shape
    e_loc, _, h = expert_W.shape
    wrows = e_loc * d

    def body(x_ref, rw_ref, ri_ref, ew_ref, out_ref,
             asm, cw1q, cw2q, ccw1q,
             p1_send, p1_recv, cw_send, cw_recv, ccw_send, ccw_recv):
        my = lax.axis_index("i")
        left = lax.rem(my - 1 + N_DEV, N_DEV)
        right = lax.rem(my + 1, N_DEV)

        bar = pltpu.get_barrier_semaphore()
        peer_offsets = [N_DEV - 1, 1, 2, STRIDE, 2 * STRIDE, 3 * STRIDE]
        for off in peer_offsets:
            peer = lax.rem(my + off, N_DEV)
            pl.semaphore_signal(
                bar, inc=1,
                device_id=(peer,), device_id_type=pl.DeviceIdType.MESH,
            )
        pl.semaphore_wait(bar, len(peer_offsets))

        xb = x_ref[:].astype(jnp.bfloat16)
        route = ri_ref[:]

        myw = ew_ref[:].astype(jnp.bfloat16).reshape(wrows, h)
        asm[0, 0:wrows, :] = myw
        asm[0, wrows:wrows + 1, :] = route.astype(jnp.bfloat16).reshape(1, m)

        def mk_p1(mm):
            return pltpu.make_async_remote_copy(
                src_ref=asm.at[0], dst_ref=asm.at[(GROUP - mm) % GROUP],
                send_sem=p1_send.at[mm - 1],
                recv_sem=p1_recv.at[(GROUP - mm) - 1],
                device_id=(lax.rem(my + STRIDE * mm, N_DEV),),
                device_id_type=pl.DeviceIdType.MESH,
            )

        for mm in range(1, GROUP):
            mk_p1(mm).start()

        ids = lax.broadcasted_iota(jnp.int32, (1, N_EXPERTS), 1)
        ids_bf = ids.astype(jnp.bfloat16)

        def chunk_contrib(w, origin):
            e0 = e_loc * origin
            m0 = (route == e0).astype(jnp.bfloat16)
            m1 = (route == e0 + 1).astype(jnp.bfloat16)
            xm = jnp.concatenate([xb * m0, xb * m1], axis=1)
            return jnp.dot(xm, w, preferred_element_type=jnp.float32)

        def hist(route_row):
            oh = (route_row.reshape(m, 1) == ids_bf).astype(jnp.float32)
            return jnp.sum(oh, axis=0, keepdims=True)

        acc = chunk_contrib(myw, my)
        oh_local = (route == ids).astype(jnp.float32)
        row = lax.broadcasted_iota(jnp.int32, (m, m), 0)
        col = lax.broadcasted_iota(jnp.int32, (m, m), 1)
        tril = (row > col).astype(jnp.float32)
        excl = jnp.dot(tril, oh_local, preferred_element_type=jnp.float32)
        prefix = jnp.zeros((1, N_EXPERTS), jnp.float32)

        def absorb(buf, sub, origin):
            c = chunk_contrib(buf[sub, 0:wrows, :], origin)
            p = jnp.where(
                origin < my, hist(buf[sub, wrows:wrows + 1, :]), 0.0
            )
            return c, p

        for j in range(1, GROUP):
            mk_p1(GROUP - j).wait_recv()

        def mk_quad(src, dst, snd, rcv, idx, tgt):
            return pltpu.make_async_remote_copy(
                src_ref=src, dst_ref=dst,
                send_sem=snd.at[idx], recv_sem=rcv.at[idx],
                device_id=(tgt,), device_id_type=pl.DeviceIdType.MESH,
            )

        cw1 = mk_quad(asm, cw1q, cw_send, cw_recv, 0, right)
        ccw1 = mk_quad(asm, ccw1q, ccw_send, ccw_recv, 0, left)
        cw2 = mk_quad(asm, cw2q, cw_send, cw_recv, 1,
                      lax.rem(my + 2, N_DEV))
        cw1.start()
        ccw1.start()
        cw2.start()

        for j in range(1, GROUP):
            c, p = absorb(asm, j, lax.rem(my + STRIDE * j, N_DEV))
            acc, prefix = acc + c, prefix + p

        cw1.wait_recv()
        for j in range(GROUP):
            c, p = absorb(cw1q, j, lax.rem(my - 1 + STRIDE * j + N_DEV, N_DEV))
            acc, prefix = acc + c, prefix + p

        ccw1.wait_recv()
        for j in range(GROUP):
            c, p = absorb(ccw1q, j, lax.rem(my + 1 + STRIDE * j, N_DEV))
            acc, prefix = acc + c, prefix + p

        cw2.wait_recv()
        for j in range(GROUP):
            c, p = absorb(cw2q, j, lax.rem(my - 2 + STRIDE * j + N_DEV, N_DEV))
            acc, prefix = acc + c, prefix + p

        for mm in range(1, GROUP):
            mk_p1(mm).wait_send()
        cw1.wait_send()
        cw2.wait_send()
        ccw1.wait_send()

        before = jnp.sum(
            oh_local * (excl + prefix), axis=1, keepdims=True
        )
        keep = (before < CAPACITY).astype(jnp.float32)
        out_ref[:] = acc * keep

    quad = pltpu.VMEM((GROUP, PACK_ROWS, h), jnp.bfloat16)
    return pl.pallas_call(
        body,
        out_shape=jax.ShapeDtypeStruct((m, h), jnp.float32),
        in_specs=[pl.BlockSpec(memory_space=pltpu.VMEM)] * 4,
        out_specs=pl.BlockSpec(memory_space=pltpu.VMEM),
        scratch_shapes=[
            quad, quad, quad, quad,
            pltpu.SemaphoreType.DMA((GROUP - 1,)),
            pltpu.SemaphoreType.DMA((GROUP - 1,)),
            pltpu.SemaphoreType.DMA((2,)),
            pltpu.SemaphoreType.DMA((2,)),
            pltpu.SemaphoreType.DMA((1,)),
            pltpu.SemaphoreType.DMA((1,)),
        ],
        compiler_params=pltpu.CompilerParams(collective_id=0),
    )(x, router_W, route_idx, expert_W)


# device time: 33512 ns/iter; 1.0069x vs baseline; 1.0069x over previous
import jax
import jax.numpy as jnp
from jax import lax
from jax.experimental import pallas as pl
from jax.experimental.pallas import tpu as pltpu

N_DEV = 16
N_EXPERTS = 32
CAPACITY = 102.0
GROUP = 4
STRIDE = N_DEV // GROUP
PACK_ROWS = 272


def kernel(x, router_W, route_idx, expert_W):
    m, d = x.shape
    e_loc, _, h = expert_W.shape
    wrows = e_loc * d

    def body(x_ref, rw_ref, ri_ref, ew_ref, out_ref,
             asm, cw1q, cw2q, ccw1q,
             p1_send, p1_recv, cw_send, cw_recv, ccw_send, ccw_recv):
        my = lax.axis_index("i")
        left = lax.rem(my - 1 + N_DEV, N_DEV)
        right = lax.rem(my + 1, N_DEV)

        bar = pltpu.get_barrier_semaphore()
        peer_offsets = [N_DEV - 1, 1, STRIDE, 2 * STRIDE, 3 * STRIDE]
        for off in peer_offsets:
            peer = lax.rem(my + off, N_DEV)
            pl.semaphore_signal(
                bar, inc=1,
                device_id=(peer,), device_id_type=pl.DeviceIdType.MESH,
            )
        pl.semaphore_wait(bar, len(peer_offsets))

        xb = x_ref[:].astype(jnp.bfloat16)
        route = ri_ref[:]

        myw = ew_ref[:].astype(jnp.bfloat16).reshape(wrows, h)
        asm[0, 0:wrows, :] = myw
        asm[0, wrows:wrows + 1, :] = route.astype(jnp.bfloat16).reshape(1, m)

        def mk_p1(mm):
            return pltpu.make_async_remote_copy(
                src_ref=asm.at[0], dst_ref=asm.at[(GROUP - mm) % GROUP],
                send_sem=p1_send.at[mm - 1],
                recv_sem=p1_recv.at[(GROUP - mm) - 1],
                device_id=(lax.rem(my + STRIDE * mm, N_DEV),),
                device_id_type=pl.DeviceIdType.MESH,
            )

        for mm in range(1, GROUP):
            mk_p1(mm).start()

        ids = lax.broadcasted_iota(jnp.int32, (1, N_EXPERTS), 1)
        ids_bf = ids.astype(jnp.bfloat16)

        def chunk_contrib(w, origin):
            e0 = e_loc * origin
            m0 = (route == e0).astype(jnp.bfloat16)
            m1 = (route == e0 + 1).astype(jnp.bfloat16)
            xm = jnp.concatenate([xb * m0, xb * m1], axis=1)
            return jnp.dot(xm, w, preferred_element_type=jnp.float32)

        def hist(route_row):
            oh = (route_row.reshape(m, 1) == ids_bf).astype(jnp.float32)
            return jnp.sum(oh, axis=0, keepdims=True)

        acc = chunk_contrib(myw, my)
        oh_local = (route == ids).astype(jnp.float32)
        row = lax.broadcasted_iota(jnp.int32, (m, m), 0)
        col = lax.broadcasted_iota(jnp.int32, (m, m), 1)
        tril = (row > col).astype(jnp.float32)
        excl = jnp.dot(tril, oh_local, preferred_element_type=jnp.float32)
        prefix = jnp.zeros((1, N_EXPERTS), jnp.float32)

        def absorb(buf, sub, origin):
            c = chunk_contrib(buf[sub, 0:wrows, :], origin)
            p = jnp.where(
                origin < my, hist(buf[sub, wrows:wrows + 1, :]), 0.0
            )
            return c, p

        for j in range(1, GROUP):
            mk_p1(GROUP - j).wait_recv()

        def mk_quad(src, dst, snd, rcv, idx, tgt):
            return pltpu.make_async_remote_copy(
                src_ref=src, dst_ref=dst,
                send_sem=snd.at[idx], recv_sem=rcv.at[idx],
                device_id=(tgt,), device_id_type=pl.DeviceIdType.MESH,
            )

        cw1 = mk_quad(asm, cw1q, cw_send, cw_recv, 0, right)
        ccw1 = mk_quad(asm, ccw1q, ccw_send, ccw_recv, 0, left)
        cw1.start()
        ccw1.start()

        for j in range(1, GROUP):
            c, p = absorb(asm, j, lax.rem(my + STRIDE * j, N_DEV))
            acc, prefix = acc + c, prefix + p

        cw1.wait_recv()
        cw2 = mk_quad(cw1q, cw2q, cw_send, cw_recv, 1, right)
        cw2.start()
        for j in range(GROUP):
            c, p = absorb(cw1q, j, lax.rem(my - 1 + STRIDE * j + N_DEV, N_DEV))
            acc, prefix = acc + c, prefix + p

        ccw1.wait_recv()
        for j in range(GROUP):
            c, p = absorb(ccw1q, j, lax.rem(my + 1 + STRIDE * j, N_DEV))
            acc, prefix = acc + c, prefix + p

        cw2.wait_recv()
        for j in range(GROUP):
            c, p = absorb(cw2q, j, lax.rem(my - 2 + STRIDE * j + N_DEV, N_DEV))
            acc, prefix = acc + c, prefix + p

        for mm in range(1, GROUP):
            mk_p1(mm).wait_send()
        cw1.wait_send()
        cw2.wait_send()
        ccw1.wait_send()

        before = jnp.sum(
            oh_local * (excl + prefix), axis=1, keepdims=True
        )
        keep = (before < CAPACITY).astype(jnp.float32)
        out_ref[:] = acc * keep

    quad = pltpu.VMEM((GROUP, PACK_ROWS, h), jnp.bfloat16)
    return pl.pallas_call(
        body,
        out_shape=jax.ShapeDtypeStruct((m, h), jnp.float32),
        in_specs=[pl.BlockSpec(memory_space=pltpu.VMEM)] * 4,
        out_specs=pl.BlockSpec(memory_space=pltpu.VMEM),
        scratch_shapes=[
            quad, quad, quad, quad,
            pltpu.SemaphoreType.DMA((GROUP - 1,)),
            pltpu.SemaphoreType.DMA((GROUP - 1,)),
            pltpu.SemaphoreType.DMA((2,)),
            pltpu.SemaphoreType.DMA((2,)),
            pltpu.SemaphoreType.DMA((1,)),
            pltpu.SemaphoreType.DMA((1,)),
        ],
        compiler_params=pltpu.CompilerParams(collective_id=0),
    )(x, router_W, route_idx, expert_W)
